# Initial kernel scaffold; baseline (speedup 1.0000x reference)
import jax
import jax.numpy as jnp
from jax.experimental import pallas as pl


def kernel(hidden_states, qr, positions, wq_b, wk, k_gamma, k_beta, w_proj):
    raise NotImplementedError("write your pallas kernel here")


# speedup vs baseline: 1.1230x; 1.1230x over previous
import functools
import jax
import jax.numpy as jnp
from jax.experimental import pallas as pl
from jax.experimental.pallas import tpu as pltpu

T = 2048
HIDDEN = 4096
Q_LORA = 1536
N_HEAD = 32
HEAD_DIM = 128
ROPE_DIM = 64
HALF = ROPE_DIM // 2
TOPK = 512
SOFTMAX_SCALE = HEAD_DIM ** -0.5
HEAD_SCALE = N_HEAD ** -0.5

BT = 256


def _proj_body(qr_ref, hs_ref, wqb_ref, wk_ref, gam_ref, bet_ref, wp_ref,
               cos_ref, sin_ref, qf_ref, kf_ref, w_ref):
    q = jnp.dot(qr_ref[...], wqb_ref[...], preferred_element_type=jnp.float32)
    q3 = q.reshape(BT, N_HEAD, HEAD_DIM)
    c = cos_ref[...][:, None, :]
    s = sin_ref[...][:, None, :]
    x1 = q3[..., :HALF]
    x2 = q3[..., HALF:ROPE_DIM]
    qf3 = jnp.concatenate([x1 * c - x2 * s, x2 * c + x1 * s, q3[..., ROPE_DIM:]],
                          axis=-1)
    qf_ref[...] = qf3.reshape(BT, N_HEAD * HEAD_DIM).astype(jnp.bfloat16)

    kl = jnp.dot(hs_ref[...], wk_ref[...], preferred_element_type=jnp.float32)
    mu = jnp.mean(kl, axis=-1, keepdims=True)
    var = jnp.mean((kl - mu) ** 2, axis=-1, keepdims=True)
    kn = (kl - mu) / jnp.sqrt(var + 1e-6) * gam_ref[...] + bet_ref[...]
    c2 = cos_ref[...]
    s2 = sin_ref[...]
    k1 = kn[:, :HALF]
    k2 = kn[:, HALF:ROPE_DIM]
    kf_ref[...] = jnp.concatenate([k1 * c2 - k2 * s2, k2 * c2 + k1 * s2,
                                   kn[:, ROPE_DIM:]], axis=-1).astype(jnp.bfloat16)

    w = jnp.dot(hs_ref[...], wp_ref[...], preferred_element_type=jnp.float32)
    w_ref[...] = ((w * SOFTMAX_SCALE) * HEAD_SCALE).astype(jnp.bfloat16)


def _scores_body(qf_ref, kf_ref, w_ref, posr_ref, posc_ref, out_ref):
    kf = kf_ref[...]
    w = w_ref[...].astype(jnp.float32)
    groups = []
    for g in range(0, N_HEAD, 8):
        acc = None
        for h in range(g, g + 8):
            qh = qf_ref[:, h * HEAD_DIM:(h + 1) * HEAD_DIM]
            lg = jax.lax.dot_general(qh, kf, (((1,), (1,)), ((), ())),
                                     preferred_element_type=jnp.float32)
            lgb = jnp.maximum(lg, 0.0).astype(jnp.bfloat16).astype(jnp.float32)
            t = lgb * w[:, h:h + 1]
            acc = t if acc is None else acc + t
        groups.append(acc)
    acc = groups[0]
    for t in groups[1:]:
        acc = acc + t
    msk = posc_ref[...] <= posr_ref[...]
    out_ref[...] = jnp.where(msk, acc, jnp.float32(-1e30))


def _compute_scores(hidden_states, qr, positions, wq_b, wk, k_gamma, k_beta,
                    w_proj):
    inv_freq = 1.0 / (10000.0 ** (jnp.arange(0, HALF, dtype=jnp.float32) / HALF))
    freqs = positions.astype(jnp.float32)[:, None] * inv_freq[None, :]
    cos = jnp.cos(freqs)
    sin = jnp.sin(freqs)

    nblk = T // BT
    qf, kf, w = pl.pallas_call(
        _proj_body,
        grid=(nblk,),
        in_specs=[
            pl.BlockSpec((BT, Q_LORA), lambda i: (i, 0)),
            pl.BlockSpec((BT, HIDDEN), lambda i: (i, 0)),
            pl.BlockSpec((Q_LORA, N_HEAD * HEAD_DIM), lambda i: (0, 0)),
            pl.BlockSpec((HIDDEN, HEAD_DIM), lambda i: (0, 0)),
            pl.BlockSpec((1, HEAD_DIM), lambda i: (0, 0)),
            pl.BlockSpec((1, HEAD_DIM), lambda i: (0, 0)),
            pl.BlockSpec((HIDDEN, N_HEAD), lambda i: (0, 0)),
            pl.BlockSpec((BT, HALF), lambda i: (i, 0)),
            pl.BlockSpec((BT, HALF), lambda i: (i, 0)),
        ],
        out_specs=[
            pl.BlockSpec((BT, N_HEAD * HEAD_DIM), lambda i: (i, 0)),
            pl.BlockSpec((BT, HEAD_DIM), lambda i: (i, 0)),
            pl.BlockSpec((BT, N_HEAD), lambda i: (i, 0)),
        ],
        out_shape=[
            jax.ShapeDtypeStruct((T, N_HEAD * HEAD_DIM), jnp.bfloat16),
            jax.ShapeDtypeStruct((T, HEAD_DIM), jnp.bfloat16),
            jax.ShapeDtypeStruct((T, N_HEAD), jnp.bfloat16),
        ],
        compiler_params=pltpu.CompilerParams(
            vmem_limit_bytes=100 * 1024 * 1024),
    )(qr, hidden_states, wq_b, wk, k_gamma.reshape(1, HEAD_DIM),
      k_beta.reshape(1, HEAD_DIM), w_proj, cos, sin)

    scores = pl.pallas_call(
        _scores_body,
        grid=(nblk,),
        in_specs=[
            pl.BlockSpec((BT, N_HEAD * HEAD_DIM), lambda i: (i, 0)),
            pl.BlockSpec((T, HEAD_DIM), lambda i: (0, 0)),
            pl.BlockSpec((BT, N_HEAD), lambda i: (i, 0)),
            pl.BlockSpec((BT, 1), lambda i: (i, 0)),
            pl.BlockSpec((1, T), lambda i: (0, 0)),
        ],
        out_specs=pl.BlockSpec((BT, T), lambda i: (i, 0)),
        out_shape=jax.ShapeDtypeStruct((T, T), jnp.float32),
        compiler_params=pltpu.CompilerParams(
            vmem_limit_bytes=100 * 1024 * 1024),
    )(qf, kf, w, positions.reshape(T, 1), positions.reshape(1, T))
    return scores


def kernel(hidden_states, qr, positions, wq_b, wk, k_gamma, k_beta, w_proj):
    scores = _compute_scores(hidden_states, qr, positions, wq_b, wk, k_gamma,
                             k_beta, w_proj)
    topk_vals, topk_idx = jax.lax.top_k(scores, TOPK)
    return topk_vals, topk_idx.astype(jnp.int32)
